# baseline (speedup 1.0000x reference)
import jax
import jax.numpy as jnp
from jax import lax
from jax.experimental import pallas as pl
from jax.experimental.pallas import tpu as pltpu
from jax.experimental.pallas import tpu_sc as plsc

NC, NS, L = 2, 16, 16
NW = NC * NS
SEQ, HID = 4096, 2048
MAXE = 64
K = 8
SEQ_SC = 1024
RPW = SEQ_SC // NW
CHUNK = 16
NCHUNK = RPW // CHUNK
NBUF = 2
TCBLK = 512
NBLK = (SEQ - SEQ_SC) // TCBLK
BIG = 2**30


def _lanes():
    return lax.broadcasted_iota(jnp.int32, (L,), 0)


def _stage1_body(ev, vals, idxs, buf0, buf1, norms, stage_v, stage_i,
                 sem0, sem1):
    cid = lax.axis_index("c")
    sid = lax.axis_index("s")
    wid = sid * NC + cid
    base = wid * RPW
    lanes = _lanes()

    bufs = (buf0, buf1)
    sems = (sem0, sem1)
    copies = [None] * NBUF
    for c in range(min(NBUF, NCHUNK)):
        copies[c] = pltpu.async_copy(
            ev.at[pl.ds(base + c * CHUNK, CHUNK), :], bufs[c], sems[c])
    for c in range(NCHUNK):
        copies[c % NBUF].wait()
        if c + NBUF < NCHUNK:
            copies[c % NBUF] = pltpu.async_copy(
                ev.at[pl.ds(base + (c + NBUF) * CHUNK, CHUNK), :],
                bufs[c % NBUF], sems[c % NBUF])
        buf = bufs[c % NBUF]

        def row_body(r, sums):
            def col_body(j, accs):
                a0, a1, a2, a3 = accs
                o = j * (8 * L)
                for u in range(8):
                    v = buf[r, pl.ds(o + u * L, L)]
                    if u % 4 == 0:
                        a0 = a0 + v * v
                    elif u % 4 == 1:
                        a1 = a1 + v * v
                    elif u % 4 == 2:
                        a2 = a2 + v * v
                    else:
                        a3 = a3 + v * v
                return a0, a1, a2, a3

            z = jnp.zeros((L,), jnp.float32)
            a0, a1, a2, a3 = lax.fori_loop(0, HID // (8 * L), col_body,
                                           (z, z, z, z))
            tot = jnp.sum((a0 + a1) + (a2 + a3))
            return jnp.where(lanes == r, tot, sums)

        sums = lax.fori_loop(0, CHUNK, row_body, jnp.zeros((L,), jnp.float32))
        norms[pl.ds(c * CHUNK, CHUNK)] = sums

    cval = jnp.full((L,), -1.0, jnp.float32)
    cidx = jnp.full((L,), BIG, jnp.int32)
    for t in range(K):
        def amax(k, carry):
            rv, ri = carry
            v = norms[pl.ds(k * L, L)]
            gi = base + k * L + lanes
            upd = (v > rv) | ((v == rv) & (gi < ri))
            return jnp.where(upd, v, rv), jnp.where(upd, gi, ri)

        rv, ri = lax.fori_loop(0, RPW // L, amax,
                               (jnp.full((L,), -2.0, jnp.float32),
                                jnp.full((L,), BIG, jnp.int32)))
        mv = jnp.max(rv)
        gv = jnp.min(jnp.where(rv == mv, ri, BIG))
        cval = jnp.where(lanes == t, mv, cval)
        cidx = jnp.where(lanes == t, gv, cidx)
        plsc.store_scatter(norms, [jnp.full((L,), gv - base, jnp.int32)],
                           jnp.full((L,), -1.0, jnp.float32),
                           mask=lanes == 0)

    stage_v[...] = cval
    stage_i[...] = cidx
    pltpu.sync_copy(stage_v, vals.at[pl.ds(wid * L, L)])
    pltpu.sync_copy(stage_i, idxs.at[pl.ds(wid * L, L)])


_stage1 = pl.kernel(
    _stage1_body,
    out_type=(jax.ShapeDtypeStruct((NW * L,), jnp.float32),
              jax.ShapeDtypeStruct((NW * L,), jnp.int32)),
    mesh=plsc.VectorSubcoreMesh(core_axis_name="c", subcore_axis_name="s", num_cores=NC),
    compiler_params=pltpu.CompilerParams(needs_layout_passes=False, skip_device_barrier=True),
    scratch_types=[
        pltpu.VMEM((CHUNK, HID), jnp.float32),
        pltpu.VMEM((CHUNK, HID), jnp.float32),
        pltpu.VMEM((RPW,), jnp.float32),
        pltpu.VMEM((L,), jnp.float32),
        pltpu.VMEM((L,), jnp.int32),
        pltpu.SemaphoreType.DMA,
        pltpu.SemaphoreType.DMA,
    ],
)

def _tcnorms_body(ev_ref, err_ref, nout_ref, fill_ref):
    x = ev_ref[...]
    nout_ref[...] = jnp.sum(x * x, axis=1).reshape(1, 1, TCBLK)

    @pl.when(pl.program_id(0) == 0)
    def _():
        fill_ref[...] = err_ref[...]


_tcnorms = pl.pallas_call(
    _tcnorms_body,
    grid=(NBLK,),
    in_specs=[pl.BlockSpec((TCBLK, HID), lambda g: (g + SEQ_SC // TCBLK, 0)),
              pl.BlockSpec((MAXE, HID), lambda g: (0, 0))],
    out_specs=[pl.BlockSpec((1, 1, TCBLK), lambda g: (g, 0, 0)),
               pl.BlockSpec((MAXE, HID), lambda g: (0, 0))],
    out_shape=[jax.ShapeDtypeStruct((NBLK, 1, TCBLK), jnp.float32),
               jax.ShapeDtypeStruct((MAXE, HID), jnp.float32)],
)


def _tcmerge_body(fill_ref, ev_ref, scv_ref, sci_ref, tcn_ref, out_ref,
                  rows, sem):
    V1 = scv_ref[...]
    I1 = sci_ref[...]
    V2 = tcn_ref[...]
    I2 = (SEQ_SC
          + lax.broadcasted_iota(jnp.int32, V2.shape, 0) * V2.shape[1]
          + lax.broadcasted_iota(jnp.int32, V2.shape, 1))
    copies = []
    for t in range(K):
        mv = jnp.maximum(jnp.max(V1), jnp.max(V2))
        sel = jnp.minimum(jnp.min(jnp.where(V1 == mv, I1, BIG)),
                          jnp.min(jnp.where(V2 == mv, I2, BIG)))
        V1 = jnp.where(I1 == sel, -2.0, V1)
        V2 = jnp.where(I2 == sel, -2.0, V2)
        c = pltpu.make_async_copy(ev_ref.at[pl.ds(sel, 1), :],
                                  rows.at[pl.ds(t, 1), :], sem)
        c.start()
        copies.append(c)
    for c in copies:
        c.wait()
    out_ref[...] = rows[...]


_tcmerge = pl.pallas_call(
    _tcmerge_body,
    grid=(1,),
    in_specs=[
        pl.BlockSpec(memory_space=pl.ANY),
        pl.BlockSpec(memory_space=pl.ANY),
        pl.BlockSpec((NW * L // 128, 128), lambda g: (0, 0)),
        pl.BlockSpec((NW * L // 128, 128), lambda g: (0, 0)),
        pl.BlockSpec(((SEQ - SEQ_SC) // 128, 128), lambda g: (0, 0)),
    ],
    out_specs=pl.BlockSpec((K, HID), lambda g: (0, 0)),
    out_shape=jax.ShapeDtypeStruct((MAXE, HID), jnp.float32),
    input_output_aliases={0: 0},
    scratch_shapes=[pltpu.VMEM((K, HID), jnp.float32),
                    pltpu.SemaphoreType.DMA],
)


@jax.jit
def kernel(error_vectors, errors):
    ev = error_vectors.reshape(4 * SEQ, HID)
    vals, idxs = _stage1(ev)
    norms_hi, fill = _tcnorms(ev, errors)
    return _tcmerge(fill, ev,
                    vals.reshape(-1, 128), idxs.reshape(-1, 128),
                    norms_hi.reshape(-1, 128))
